# baseline (speedup 1.0000x reference)
import functools

import jax
import jax.numpy as jnp
from jax import lax
from jax.experimental import pallas as pl
from jax.experimental.pallas import tpu as pltpu
from jax.experimental.pallas import tpu_sc as plsc

VOCAB_SIZE = 1000000
EMBED_DIM = 64
BATCH = 16384
K_NEG = 20

NUM_CORES = 2
NUM_SUBCORES = 16
NUM_WORKERS = NUM_CORES * NUM_SUBCORES

B_PER_W = BATCH // NUM_WORKERS
CHUNK = 32
N_CHUNKS = B_PER_W // CHUNK
SCORES_PER_ITEM = K_NEG + 1
CHUNK_SCORES = CHUNK * SCORES_PER_ITEM
GATHER_MAX = 128
NEG_PER_W = B_PER_W * K_NEG
NEG_PER_CHUNK = CHUNK * K_NEG


def _derive_indices(idx_ref, ridx_ref, n):
    def body(g, carry):
        v = idx_ref[pl.ds(g * 16, 16)]
        ridx_ref[pl.ds(g * 16, 16)] = lax.shift_right_logical(v, 1)
        idx_ref[pl.ds(g * 16, 16)] = (v & 1) * 64
        return carry
    lax.fori_loop(0, n // 16, body, 0)


def _sc_scores_kernel(cen_w, ctx_w, neg_w, cen_emb, ctx_emb, scores_out,
                      cen_idx, pos_idx, neg_idx,
                      cen_ridx, pos_ridx, neg_ridx,
                      cen_rows, pos_rows, neg_rows, partials, scores, sem):
    wid = lax.axis_index("s") * NUM_CORES + lax.axis_index("c")
    base = wid * B_PER_W

    pltpu.sync_copy(cen_w.at[pl.ds(base, B_PER_W)],
                    cen_idx.at[pl.ds(0, B_PER_W)])
    pltpu.sync_copy(ctx_w.at[pl.ds(base, B_PER_W)],
                    pos_idx.at[pl.ds(0, B_PER_W)])
    pltpu.sync_copy(neg_w.at[pl.ds(base * K_NEG, NEG_PER_W)],
                    neg_idx.at[pl.ds(0, NEG_PER_W)])
    _derive_indices(cen_idx, cen_ridx, B_PER_W)
    _derive_indices(pos_idx, pos_ridx, B_PER_W)
    _derive_indices(neg_idx, neg_ridx, NEG_PER_W)

    def chunk_body(c, chunk_carry):
        copies = [
            pltpu.async_copy(
                cen_emb.at[cen_ridx.at[pl.ds(c * CHUNK, CHUNK)]],
                cen_rows, sem),
            pltpu.async_copy(
                ctx_emb.at[pos_ridx.at[pl.ds(c * CHUNK, CHUNK)]],
                pos_rows, sem),
        ]
        for g in range(NEG_PER_CHUNK // GATHER_MAX):
            copies.append(pltpu.async_copy(
                ctx_emb.at[neg_ridx.at[pl.ds(c * NEG_PER_CHUNK
                                             + g * GATHER_MAX, GATHER_MAX)]],
                neg_rows.at[pl.ds(g * GATHER_MAX, GATHER_MAX)], sem))
        for cp in copies:
            cp.wait()

        def item_body(it, carry):
            coff = cen_idx[pl.ds(c * CHUNK + it, 16)][0]
            cvec = [cen_rows[it, pl.ds(coff + 16 * j, 16)] for j in range(4)]
            poff = pos_idx[pl.ds(c * CHUNK + it, 16)][0]
            acc = cvec[0] * pos_rows[it, pl.ds(poff, 16)]
            for j in range(1, 4):
                acc = acc + cvec[j] * pos_rows[it, pl.ds(poff + 16 * j, 16)]
            pbase = it * SCORES_PER_ITEM * 16
            partials[pl.ds(pbase, 16)] = acc
            for k in range(K_NEG):
                r = it * K_NEG + k
                noff = neg_idx[pl.ds(c * NEG_PER_CHUNK + r, 16)][0]
                acc = cvec[0] * neg_rows[r, pl.ds(noff, 16)]
                for j in range(1, 4):
                    acc = acc + cvec[j] * neg_rows[r, pl.ds(noff + 16 * j, 16)]
                partials[pl.ds(pbase + (1 + k) * 16, 16)] = -acc
            return carry

        lax.fori_loop(0, CHUNK, item_body, 0)

        ivec = lax.iota(jnp.int32, 16) * 16

        def group_body(grp, carry):
            acc = plsc.load_gather(partials, [ivec + grp * 256])
            for d in range(1, 16):
                acc = acc + plsc.load_gather(partials, [ivec + (grp * 256 + d)])
            scores[pl.ds(grp * 16, 16)] = acc
            return carry

        lax.fori_loop(0, CHUNK_SCORES // 16, group_body, 0)
        pltpu.sync_copy(
            scores,
            scores_out.at[pl.ds(base * SCORES_PER_ITEM + c * CHUNK_SCORES,
                                CHUNK_SCORES)])
        return chunk_carry

    lax.fori_loop(0, N_CHUNKS, chunk_body, 0)


@functools.partial(
    pl.kernel,
    out_type=jax.ShapeDtypeStruct((BATCH * SCORES_PER_ITEM,), jnp.float32),
    mesh=plsc.VectorSubcoreMesh(core_axis_name="c", subcore_axis_name="s"),
    compiler_params=pltpu.CompilerParams(needs_layout_passes=False,
                                         use_tc_tiling_on_sc=True),
    scratch_types=[
        pltpu.VMEM((B_PER_W + 16,), jnp.int32),
        pltpu.VMEM((B_PER_W + 16,), jnp.int32),
        pltpu.VMEM((NEG_PER_W + 16,), jnp.int32),
        pltpu.VMEM((B_PER_W,), jnp.int32),
        pltpu.VMEM((B_PER_W,), jnp.int32),
        pltpu.VMEM((NEG_PER_W,), jnp.int32),
        pltpu.VMEM((CHUNK, 128), jnp.float32),
        pltpu.VMEM((CHUNK, 128), jnp.float32),
        pltpu.VMEM((NEG_PER_CHUNK, 128), jnp.float32),
        pltpu.VMEM((CHUNK_SCORES * 16,), jnp.float32),
        pltpu.VMEM((CHUNK_SCORES,), jnp.float32),
        pltpu.SemaphoreType.DMA,
    ],
)
def _sc_scores(*args):
    _sc_scores_kernel(*args)


def _tc_loss_kernel(s_ref, o_ref):
    x = s_ref[...]
    y = -jnp.log(jax.nn.sigmoid(x) + 1e-10)
    o_ref[0, 0] = jnp.sum(y) / BATCH


def kernel(center_words, context_words, negative_samples, center_emb,
           context_emb):
    cen_w = center_words.astype(jnp.int32)
    ctx_w = context_words.astype(jnp.int32)
    neg_w = negative_samples.astype(jnp.int32).reshape(-1)
    cen2 = center_emb.reshape(VOCAB_SIZE // 2, 2 * EMBED_DIM)
    ctx2 = context_emb.reshape(VOCAB_SIZE // 2, 2 * EMBED_DIM)
    scores = _sc_scores(cen_w, ctx_w, neg_w, cen2, ctx2)
    scores2d = scores.reshape(BATCH * SCORES_PER_ITEM // 128, 128)
    loss = pl.pallas_call(
        _tc_loss_kernel,
        out_shape=jax.ShapeDtypeStruct((1, 1), jnp.float32),
        in_specs=[pl.BlockSpec(memory_space=pltpu.VMEM)],
        out_specs=pl.BlockSpec(memory_space=pltpu.SMEM),
    )(scores2d)
    return loss[0, 0]


# speedup vs baseline: 1.9335x; 1.9335x over previous
import functools

import jax
import jax.numpy as jnp
from jax import lax
from jax.experimental import pallas as pl
from jax.experimental.pallas import tpu as pltpu
from jax.experimental.pallas import tpu_sc as plsc

VOCAB_SIZE = 1000000
EMBED_DIM = 64
BATCH = 16384
K_NEG = 20

NUM_CORES = 2
NUM_SUBCORES = 16
NUM_WORKERS = NUM_CORES * NUM_SUBCORES

B_PER_W = BATCH // NUM_WORKERS
CHUNK = 32
N_CHUNKS = B_PER_W // CHUNK
SCORES_PER_ITEM = K_NEG + 1
CHUNK_SCORES = CHUNK * SCORES_PER_ITEM
GATHER_MAX = 128
NEG_PER_W = B_PER_W * K_NEG
NEG_PER_CHUNK = CHUNK * K_NEG

REPACK_BLK = 2048


def _repack_body(cen_ref, ctx_ref, o_ref):
    xc = cen_ref[...]
    xx = ctx_ref[...]
    xp = jnp.concatenate([xc, xx], axis=0)
    o_ref[...] = jnp.transpose(xp, (1, 0))


def _repack(cen_t, ctx_t):
    return pl.pallas_call(
        _repack_body,
        grid=(VOCAB_SIZE // REPACK_BLK,),
        in_specs=[
            pl.BlockSpec((EMBED_DIM, REPACK_BLK), lambda i: (0, i)),
            pl.BlockSpec((EMBED_DIM, REPACK_BLK), lambda i: (0, i)),
        ],
        out_specs=pl.BlockSpec((REPACK_BLK, 128), lambda i: (i, 0)),
        out_shape=jax.ShapeDtypeStruct((VOCAB_SIZE, 128), jnp.float32),
    )(cen_t, ctx_t)


def _sc_scores_kernel(cen_w, ctx_w, neg_w, tbl, scores_out,
                      cen_idx, pos_idx, neg_idx,
                      cen_rows, pos_rows, neg_rows, partials, scores, sem):
    wid = lax.axis_index("s") * NUM_CORES + lax.axis_index("c")
    base = wid * B_PER_W

    pltpu.sync_copy(cen_w.at[pl.ds(base, B_PER_W)], cen_idx)
    pltpu.sync_copy(ctx_w.at[pl.ds(base, B_PER_W)], pos_idx)
    pltpu.sync_copy(neg_w.at[pl.ds(base * K_NEG, NEG_PER_W)], neg_idx)

    def chunk_body(c, chunk_carry):
        copies = [
            pltpu.async_copy(
                tbl.at[cen_idx.at[pl.ds(c * CHUNK, CHUNK)]], cen_rows, sem),
            pltpu.async_copy(
                tbl.at[pos_idx.at[pl.ds(c * CHUNK, CHUNK)]], pos_rows, sem),
        ]
        for g in range(NEG_PER_CHUNK // GATHER_MAX):
            copies.append(pltpu.async_copy(
                tbl.at[neg_idx.at[pl.ds(c * NEG_PER_CHUNK
                                        + g * GATHER_MAX, GATHER_MAX)]],
                neg_rows.at[pl.ds(g * GATHER_MAX, GATHER_MAX)], sem))
        for cp in copies:
            cp.wait()

        def item_body(it, carry):
            cvec = [cen_rows[it, pl.ds(16 * j, 16)] for j in range(4)]
            acc = cvec[0] * pos_rows[it, pl.ds(64, 16)]
            for j in range(1, 4):
                acc = acc + cvec[j] * pos_rows[it, pl.ds(64 + 16 * j, 16)]
            pbase = it * SCORES_PER_ITEM * 16
            partials[pl.ds(pbase, 16)] = acc
            for k in range(K_NEG):
                r = it * K_NEG + k
                acc = cvec[0] * neg_rows[r, pl.ds(64, 16)]
                for j in range(1, 4):
                    acc = acc + cvec[j] * neg_rows[r, pl.ds(64 + 16 * j, 16)]
                partials[pl.ds(pbase + (1 + k) * 16, 16)] = -acc
            return carry

        lax.fori_loop(0, CHUNK, item_body, 0)

        ivec = lax.iota(jnp.int32, 16) * 16

        def group_body(grp, carry):
            acc = plsc.load_gather(partials, [ivec + grp * 256])
            for d in range(1, 16):
                acc = acc + plsc.load_gather(partials, [ivec + (grp * 256 + d)])
            scores[pl.ds(grp * 16, 16)] = acc
            return carry

        lax.fori_loop(0, CHUNK_SCORES // 16, group_body, 0)
        pltpu.sync_copy(
            scores,
            scores_out.at[pl.ds(base * SCORES_PER_ITEM + c * CHUNK_SCORES,
                                CHUNK_SCORES)])
        return chunk_carry

    lax.fori_loop(0, N_CHUNKS, chunk_body, 0)


@functools.partial(
    pl.kernel,
    out_type=jax.ShapeDtypeStruct((BATCH * SCORES_PER_ITEM,), jnp.float32),
    mesh=plsc.VectorSubcoreMesh(core_axis_name="c", subcore_axis_name="s"),
    compiler_params=pltpu.CompilerParams(needs_layout_passes=False,
                                         use_tc_tiling_on_sc=True),
    scratch_types=[
        pltpu.VMEM((B_PER_W,), jnp.int32),
        pltpu.VMEM((B_PER_W,), jnp.int32),
        pltpu.VMEM((NEG_PER_W,), jnp.int32),
        pltpu.VMEM((CHUNK, 128), jnp.float32),
        pltpu.VMEM((CHUNK, 128), jnp.float32),
        pltpu.VMEM((NEG_PER_CHUNK, 128), jnp.float32),
        pltpu.VMEM((CHUNK_SCORES * 16,), jnp.float32),
        pltpu.VMEM((CHUNK_SCORES,), jnp.float32),
        pltpu.SemaphoreType.DMA,
    ],
)
def _sc_scores(*args):
    _sc_scores_kernel(*args)


def _tc_loss_kernel(s_ref, o_ref):
    x = s_ref[...]
    y = -jnp.log(jax.nn.sigmoid(x) + 1e-10)
    o_ref[0, 0] = jnp.sum(y) / BATCH


def kernel(center_words, context_words, negative_samples, center_emb,
           context_emb):
    cen_w = center_words.astype(jnp.int32)
    ctx_w = context_words.astype(jnp.int32)
    neg_w = negative_samples.astype(jnp.int32).reshape(-1)
    merged = _repack(center_emb.T, context_emb.T)
    scores = _sc_scores(cen_w, ctx_w, neg_w, merged)
    scores2d = scores.reshape(BATCH * SCORES_PER_ITEM // 128, 128)
    loss = pl.pallas_call(
        _tc_loss_kernel,
        out_shape=jax.ShapeDtypeStruct((1, 1), jnp.float32),
        in_specs=[pl.BlockSpec(memory_space=pltpu.VMEM)],
        out_specs=pl.BlockSpec(memory_space=pltpu.SMEM),
    )(scores2d)
    return loss[0, 0]


# speedup vs baseline: 2.6507x; 1.3709x over previous
import functools

import jax
import jax.numpy as jnp
from jax import lax
from jax.experimental import pallas as pl
from jax.experimental.pallas import tpu as pltpu
from jax.experimental.pallas import tpu_sc as plsc

VOCAB_SIZE = 1000000
EMBED_DIM = 64
BATCH = 16384
K_NEG = 20

NUM_CORES = 2
NUM_SUBCORES = 16
NUM_WORKERS = NUM_CORES * NUM_SUBCORES

B_PER_W = BATCH // NUM_WORKERS
CHUNK = 16
N_CHUNKS = B_PER_W // CHUNK
SCORES_PER_ITEM = K_NEG + 1
CHUNK_SCORES = CHUNK * SCORES_PER_ITEM
GATHER_MAX = 128
NEG_PER_W = B_PER_W * K_NEG
NEG_PER_CHUNK = CHUNK * K_NEG

REPACK_BLK = 4096


def _repack_body(cen_ref, ctx_ref, o_ref):
    xc = cen_ref[...]
    xx = ctx_ref[...]
    xp = jnp.concatenate([xc, xx], axis=0)
    o_ref[...] = jnp.transpose(xp, (1, 0))


def _repack(cen_t, ctx_t):
    return pl.pallas_call(
        _repack_body,
        grid=(VOCAB_SIZE // REPACK_BLK,),
        in_specs=[
            pl.BlockSpec((EMBED_DIM, REPACK_BLK), lambda i: (0, i)),
            pl.BlockSpec((EMBED_DIM, REPACK_BLK), lambda i: (0, i)),
        ],
        out_specs=pl.BlockSpec((REPACK_BLK, 128), lambda i: (i, 0)),
        out_shape=jax.ShapeDtypeStruct((VOCAB_SIZE, 128), jnp.float32),
    )(cen_t, ctx_t)


def _sc_scores_kernel(cen_w, ctx_w, neg_w, tbl, scores_out,
                      cen_idx, pos_idx, neg_idx,
                      cen_rows0, pos_rows0, neg_rows0,
                      cen_rows1, pos_rows1, neg_rows1,
                      partials, scores, sem0, sem1):
    wid = lax.axis_index("s") * NUM_CORES + lax.axis_index("c")
    base = wid * B_PER_W
    bufs = ((cen_rows0, pos_rows0, neg_rows0, sem0),
            (cen_rows1, pos_rows1, neg_rows1, sem1))

    pltpu.sync_copy(cen_w.at[pl.ds(base, B_PER_W)], cen_idx)
    pltpu.sync_copy(ctx_w.at[pl.ds(base, B_PER_W)], pos_idx)
    pltpu.sync_copy(neg_w.at[pl.ds(base * K_NEG, NEG_PER_W)], neg_idx)

    def fire(c, buf):
        cen_rows, pos_rows, neg_rows, sem = buf
        pltpu.async_copy(
            tbl.at[cen_idx.at[pl.ds(c * CHUNK, CHUNK)]], cen_rows, sem)
        pltpu.async_copy(
            tbl.at[pos_idx.at[pl.ds(c * CHUNK, CHUNK)]], pos_rows, sem)
        for g in range(NEG_PER_CHUNK // GATHER_MAX):
            pltpu.async_copy(
                tbl.at[neg_idx.at[pl.ds(c * NEG_PER_CHUNK
                                        + g * GATHER_MAX, GATHER_MAX)]],
                neg_rows.at[pl.ds(g * GATHER_MAX, GATHER_MAX)], sem)
        rem = NEG_PER_CHUNK % GATHER_MAX
        if rem:
            g0 = (NEG_PER_CHUNK // GATHER_MAX) * GATHER_MAX
            pltpu.async_copy(
                tbl.at[neg_idx.at[pl.ds(c * NEG_PER_CHUNK + g0, rem)]],
                neg_rows.at[pl.ds(g0, rem)], sem)

    def drain(buf):
        cen_rows, pos_rows, neg_rows, sem = buf
        pltpu.make_async_copy(tbl.at[cen_idx.at[pl.ds(0, CHUNK)]],
                              cen_rows, sem).wait()
        pltpu.make_async_copy(tbl.at[pos_idx.at[pl.ds(0, CHUNK)]],
                              pos_rows, sem).wait()
        pltpu.make_async_copy(tbl.at[neg_idx.at[pl.ds(0, NEG_PER_CHUNK)]],
                              neg_rows, sem).wait()

    def compute(c, buf):
        cen_rows, pos_rows, neg_rows, _ = buf

        def item_body(it, carry):
            cvec = [cen_rows[it, pl.ds(16 * j, 16)] for j in range(4)]
            acc = cvec[0] * pos_rows[it, pl.ds(64, 16)]
            for j in range(1, 4):
                acc = acc + cvec[j] * pos_rows[it, pl.ds(64 + 16 * j, 16)]
            pbase = it * SCORES_PER_ITEM * 16
            partials[pl.ds(pbase, 16)] = acc
            for k in range(K_NEG):
                r = it * K_NEG + k
                acc = cvec[0] * neg_rows[r, pl.ds(64, 16)]
                for j in range(1, 4):
                    acc = acc + cvec[j] * neg_rows[r, pl.ds(64 + 16 * j, 16)]
                partials[pl.ds(pbase + (1 + k) * 16, 16)] = -acc
            return carry

        lax.fori_loop(0, CHUNK, item_body, 0, unroll=2)

        ivec = lax.iota(jnp.int32, 16) * 16

        def group_body(grp, carry):
            acc = plsc.load_gather(partials, [ivec + grp * 256])
            for d in range(1, 16):
                acc = acc + plsc.load_gather(partials, [ivec + (grp * 256 + d)])
            scores[pl.ds(grp * 16, 16)] = acc
            return carry

        lax.fori_loop(0, CHUNK_SCORES // 16, group_body, 0, unroll=2)
        pltpu.sync_copy(
            scores,
            scores_out.at[pl.ds(base * SCORES_PER_ITEM + c * CHUNK_SCORES,
                                CHUNK_SCORES)])

    fire(0, bufs[0])

    def pair_body(c2, carry):
        c0 = 2 * c2
        fire(c0 + 1, bufs[1])
        drain(bufs[0])
        compute(c0, bufs[0])

        @pl.when(c2 < N_CHUNKS // 2 - 1)
        def _():
            fire(c0 + 2, bufs[0])

        drain(bufs[1])
        compute(c0 + 1, bufs[1])
        return carry

    lax.fori_loop(0, N_CHUNKS // 2, pair_body, 0)


@functools.partial(
    pl.kernel,
    out_type=jax.ShapeDtypeStruct((BATCH * SCORES_PER_ITEM,), jnp.float32),
    mesh=plsc.VectorSubcoreMesh(core_axis_name="c", subcore_axis_name="s"),
    compiler_params=pltpu.CompilerParams(needs_layout_passes=False,
                                         use_tc_tiling_on_sc=True),
    scratch_types=[
        pltpu.VMEM((B_PER_W,), jnp.int32),
        pltpu.VMEM((B_PER_W,), jnp.int32),
        pltpu.VMEM((NEG_PER_W,), jnp.int32),
        pltpu.VMEM((CHUNK, 128), jnp.float32),
        pltpu.VMEM((CHUNK, 128), jnp.float32),
        pltpu.VMEM((NEG_PER_CHUNK, 128), jnp.float32),
        pltpu.VMEM((CHUNK, 128), jnp.float32),
        pltpu.VMEM((CHUNK, 128), jnp.float32),
        pltpu.VMEM((NEG_PER_CHUNK, 128), jnp.float32),
        pltpu.VMEM((CHUNK_SCORES * 16,), jnp.float32),
        pltpu.VMEM((CHUNK_SCORES,), jnp.float32),
        pltpu.SemaphoreType.DMA,
        pltpu.SemaphoreType.DMA,
    ],
)
def _sc_scores(*args):
    _sc_scores_kernel(*args)


def _tc_loss_kernel(s_ref, o_ref):
    x = s_ref[...]
    y = -jnp.log(jax.nn.sigmoid(x) + 1e-10)
    o_ref[0, 0] = jnp.sum(y) / BATCH


def kernel(center_words, context_words, negative_samples, center_emb,
           context_emb):
    cen_w = center_words.astype(jnp.int32)
    ctx_w = context_words.astype(jnp.int32)
    neg_w = negative_samples.astype(jnp.int32).reshape(-1)
    merged = _repack(center_emb.T, context_emb.T)
    scores = _sc_scores(cen_w, ctx_w, neg_w, merged)
    scores2d = scores.reshape(BATCH * SCORES_PER_ITEM // 128, 128)
    loss = pl.pallas_call(
        _tc_loss_kernel,
        out_shape=jax.ShapeDtypeStruct((1, 1), jnp.float32),
        in_specs=[pl.BlockSpec(memory_space=pltpu.VMEM)],
        out_specs=pl.BlockSpec(memory_space=pltpu.SMEM),
    )(scores2d)
    return loss[0, 0]


# speedup vs baseline: 2.9488x; 1.1125x over previous
import functools

import jax
import jax.numpy as jnp
from jax import lax
from jax.experimental import pallas as pl
from jax.experimental.pallas import tpu as pltpu
from jax.experimental.pallas import tpu_sc as plsc

VOCAB_SIZE = 1000000
EMBED_DIM = 64
BATCH = 16384
K_NEG = 20

NUM_CORES = 2
NUM_SUBCORES = 16
NUM_WORKERS = NUM_CORES * NUM_SUBCORES

B_PER_W = BATCH // NUM_WORKERS
CHUNK = 16
N_CHUNKS = B_PER_W // CHUNK
SCORES_PER_ITEM = K_NEG + 1
CHUNK_SCORES = CHUNK * SCORES_PER_ITEM
GATHER_MAX = 128
NEG_PER_W = B_PER_W * K_NEG
NEG_PER_CHUNK = CHUNK * K_NEG

REPACK_BLK = 8192


def _repack_body(cen_ref, ctx_ref, o_ref):
    xc = cen_ref[...]
    xx = ctx_ref[...]
    xp = jnp.concatenate([xc, xx], axis=0)
    o_ref[...] = jnp.transpose(xp, (1, 0))


def _repack(cen_t, ctx_t):
    return pl.pallas_call(
        _repack_body,
        grid=(VOCAB_SIZE // REPACK_BLK,),
        in_specs=[
            pl.BlockSpec((EMBED_DIM, REPACK_BLK), lambda i: (0, i)),
            pl.BlockSpec((EMBED_DIM, REPACK_BLK), lambda i: (0, i)),
        ],
        out_specs=pl.BlockSpec((REPACK_BLK, 128), lambda i: (i, 0)),
        out_shape=jax.ShapeDtypeStruct((VOCAB_SIZE, 128), jnp.float32),
    )(cen_t, ctx_t)


def _sc_scores_kernel(cen_w, ctx_w, neg_w, tbl, scores_out,
                      cen_idx, pos_idx, neg_idx,
                      cen_rows0, pos_rows0, neg_rows0,
                      cen_rows1, pos_rows1, neg_rows1,
                      partials, scores, sem0, sem1):
    wid = lax.axis_index("s") * NUM_CORES + lax.axis_index("c")
    base = wid * B_PER_W
    bufs = ((cen_rows0, pos_rows0, neg_rows0, sem0),
            (cen_rows1, pos_rows1, neg_rows1, sem1))

    pltpu.sync_copy(cen_w.at[pl.ds(base, B_PER_W)], cen_idx)
    pltpu.sync_copy(ctx_w.at[pl.ds(base, B_PER_W)], pos_idx)
    pltpu.sync_copy(neg_w.at[pl.ds(base * K_NEG, NEG_PER_W)], neg_idx)

    def fire(c, buf):
        cen_rows, pos_rows, neg_rows, sem = buf
        pltpu.async_copy(
            tbl.at[cen_idx.at[pl.ds(c * CHUNK, CHUNK)]], cen_rows, sem)
        pltpu.async_copy(
            tbl.at[pos_idx.at[pl.ds(c * CHUNK, CHUNK)]], pos_rows, sem)
        for g in range(NEG_PER_CHUNK // GATHER_MAX):
            pltpu.async_copy(
                tbl.at[neg_idx.at[pl.ds(c * NEG_PER_CHUNK
                                        + g * GATHER_MAX, GATHER_MAX)]],
                neg_rows.at[pl.ds(g * GATHER_MAX, GATHER_MAX)], sem)
        rem = NEG_PER_CHUNK % GATHER_MAX
        if rem:
            g0 = (NEG_PER_CHUNK // GATHER_MAX) * GATHER_MAX
            pltpu.async_copy(
                tbl.at[neg_idx.at[pl.ds(c * NEG_PER_CHUNK + g0, rem)]],
                neg_rows.at[pl.ds(g0, rem)], sem)

    def drain(buf):
        cen_rows, pos_rows, neg_rows, sem = buf
        pltpu.make_async_copy(tbl.at[cen_idx.at[pl.ds(0, CHUNK)]],
                              cen_rows, sem).wait()
        pltpu.make_async_copy(tbl.at[pos_idx.at[pl.ds(0, CHUNK)]],
                              pos_rows, sem).wait()
        pltpu.make_async_copy(tbl.at[neg_idx.at[pl.ds(0, NEG_PER_CHUNK)]],
                              neg_rows, sem).wait()

    def compute(c, buf):
        cen_rows, pos_rows, neg_rows, _ = buf

        def item_body(it, carry):
            cvec = [cen_rows[it, pl.ds(16 * j, 16)] for j in range(4)]
            acc = cvec[0] * pos_rows[it, pl.ds(64, 16)]
            for j in range(1, 4):
                acc = acc + cvec[j] * pos_rows[it, pl.ds(64 + 16 * j, 16)]
            pbase = it * SCORES_PER_ITEM * 16
            partials[pl.ds(pbase, 16)] = acc
            for k in range(K_NEG):
                r = it * K_NEG + k
                acc = cvec[0] * neg_rows[r, pl.ds(64, 16)]
                for j in range(1, 4):
                    acc = acc + cvec[j] * neg_rows[r, pl.ds(64 + 16 * j, 16)]
                partials[pl.ds(pbase + (1 + k) * 16, 16)] = -acc
            return carry

        lax.fori_loop(0, CHUNK, item_body, 0, unroll=2)

        ivec = lax.iota(jnp.int32, 16) * 16

        def group_body(grp, carry):
            acc = plsc.load_gather(partials, [ivec + grp * 256])
            for d in range(1, 16):
                acc = acc + plsc.load_gather(partials, [ivec + (grp * 256 + d)])
            scores[pl.ds(grp * 16, 16)] = acc
            return carry

        lax.fori_loop(0, CHUNK_SCORES // 16, group_body, 0, unroll=2)
        pltpu.sync_copy(
            scores,
            scores_out.at[pl.ds(base * SCORES_PER_ITEM + c * CHUNK_SCORES,
                                CHUNK_SCORES)])

    fire(0, bufs[0])

    def pair_body(c2, carry):
        c0 = 2 * c2
        fire(c0 + 1, bufs[1])
        drain(bufs[0])
        compute(c0, bufs[0])

        @pl.when(c2 < N_CHUNKS // 2 - 1)
        def _():
            fire(c0 + 2, bufs[0])

        drain(bufs[1])
        compute(c0 + 1, bufs[1])
        return carry

    lax.fori_loop(0, N_CHUNKS // 2, pair_body, 0)


@functools.partial(
    pl.kernel,
    out_type=jax.ShapeDtypeStruct((BATCH * SCORES_PER_ITEM,), jnp.float32),
    mesh=plsc.VectorSubcoreMesh(core_axis_name="c", subcore_axis_name="s"),
    compiler_params=pltpu.CompilerParams(needs_layout_passes=False,
                                         use_tc_tiling_on_sc=True),
    scratch_types=[
        pltpu.VMEM((B_PER_W,), jnp.int32),
        pltpu.VMEM((B_PER_W,), jnp.int32),
        pltpu.VMEM((NEG_PER_W,), jnp.int32),
        pltpu.VMEM((CHUNK, 128), jnp.float32),
        pltpu.VMEM((CHUNK, 128), jnp.float32),
        pltpu.VMEM((NEG_PER_CHUNK, 128), jnp.float32),
        pltpu.VMEM((CHUNK, 128), jnp.float32),
        pltpu.VMEM((CHUNK, 128), jnp.float32),
        pltpu.VMEM((NEG_PER_CHUNK, 128), jnp.float32),
        pltpu.VMEM((CHUNK_SCORES * 16,), jnp.float32),
        pltpu.VMEM((CHUNK_SCORES,), jnp.float32),
        pltpu.SemaphoreType.DMA,
        pltpu.SemaphoreType.DMA,
    ],
)
def _sc_scores(*args):
    _sc_scores_kernel(*args)


def _tc_loss_kernel(s_ref, o_ref):
    x = s_ref[...]
    y = -jnp.log(jax.nn.sigmoid(x) + 1e-10)
    o_ref[0, 0] = jnp.sum(y) / BATCH


def kernel(center_words, context_words, negative_samples, center_emb,
           context_emb):
    cen_w = center_words.astype(jnp.int32)
    ctx_w = context_words.astype(jnp.int32)
    neg_w = negative_samples.astype(jnp.int32).reshape(-1)
    merged = _repack(center_emb.T, context_emb.T)
    scores = _sc_scores(cen_w, ctx_w, neg_w, merged)
    scores2d = scores.reshape(BATCH * SCORES_PER_ITEM // 128, 128)
    loss = pl.pallas_call(
        _tc_loss_kernel,
        out_shape=jax.ShapeDtypeStruct((1, 1), jnp.float32),
        in_specs=[pl.BlockSpec(memory_space=pltpu.VMEM)],
        out_specs=pl.BlockSpec(memory_space=pltpu.SMEM),
    )(scores2d)
    return loss[0, 0]


# speedup vs baseline: 3.0061x; 1.0194x over previous
import functools

import jax
import jax.numpy as jnp
from jax import lax
from jax.experimental import pallas as pl
from jax.experimental.pallas import tpu as pltpu
from jax.experimental.pallas import tpu_sc as plsc

VOCAB_SIZE = 1000000
EMBED_DIM = 64
BATCH = 16384
K_NEG = 20

NUM_CORES = 2
NUM_SUBCORES = 16
NUM_WORKERS = NUM_CORES * NUM_SUBCORES

B_PER_W = BATCH // NUM_WORKERS
CHUNK = 16
N_CHUNKS = B_PER_W // CHUNK
SCORES_PER_ITEM = K_NEG + 1
CHUNK_SCORES = CHUNK * SCORES_PER_ITEM
GATHER_MAX = 128
NEG_PER_W = B_PER_W * K_NEG
NEG_PER_CHUNK = CHUNK * K_NEG

REPACK_BLK = 16384


def _repack_body(cen_ref, ctx_ref, o_ref):
    xc = cen_ref[...]
    xx = ctx_ref[...]
    xp = jnp.concatenate([xc, xx], axis=0)
    o_ref[...] = jnp.transpose(xp, (1, 0))


def _repack(cen_t, ctx_t):
    return pl.pallas_call(
        _repack_body,
        grid=(VOCAB_SIZE // REPACK_BLK,),
        in_specs=[
            pl.BlockSpec((EMBED_DIM, REPACK_BLK), lambda i: (0, i)),
            pl.BlockSpec((EMBED_DIM, REPACK_BLK), lambda i: (0, i)),
        ],
        out_specs=pl.BlockSpec((REPACK_BLK, 128), lambda i: (i, 0)),
        out_shape=jax.ShapeDtypeStruct((VOCAB_SIZE, 128), jnp.float32),
    )(cen_t, ctx_t)


def _sc_scores_kernel(cen_w, ctx_w, neg_w, tbl, scores_out,
                      cen_idx, pos_idx, neg_idx,
                      cen_rows0, pos_rows0, neg_rows0,
                      cen_rows1, pos_rows1, neg_rows1,
                      partials, scores, sem0, sem1):
    wid = lax.axis_index("s") * NUM_CORES + lax.axis_index("c")
    base = wid * B_PER_W
    bufs = ((cen_rows0, pos_rows0, neg_rows0, sem0),
            (cen_rows1, pos_rows1, neg_rows1, sem1))

    pltpu.sync_copy(cen_w.at[pl.ds(base, B_PER_W)], cen_idx)
    pltpu.sync_copy(ctx_w.at[pl.ds(base, B_PER_W)], pos_idx)
    pltpu.sync_copy(neg_w.at[pl.ds(base * K_NEG, NEG_PER_W)], neg_idx)

    def fire(c, buf):
        cen_rows, pos_rows, neg_rows, sem = buf
        pltpu.async_copy(
            tbl.at[cen_idx.at[pl.ds(c * CHUNK, CHUNK)]], cen_rows, sem)
        pltpu.async_copy(
            tbl.at[pos_idx.at[pl.ds(c * CHUNK, CHUNK)]], pos_rows, sem)
        for g in range(NEG_PER_CHUNK // GATHER_MAX):
            pltpu.async_copy(
                tbl.at[neg_idx.at[pl.ds(c * NEG_PER_CHUNK
                                        + g * GATHER_MAX, GATHER_MAX)]],
                neg_rows.at[pl.ds(g * GATHER_MAX, GATHER_MAX)], sem)
        rem = NEG_PER_CHUNK % GATHER_MAX
        if rem:
            g0 = (NEG_PER_CHUNK // GATHER_MAX) * GATHER_MAX
            pltpu.async_copy(
                tbl.at[neg_idx.at[pl.ds(c * NEG_PER_CHUNK + g0, rem)]],
                neg_rows.at[pl.ds(g0, rem)], sem)

    def drain(buf):
        cen_rows, pos_rows, neg_rows, sem = buf
        pltpu.make_async_copy(tbl.at[cen_idx.at[pl.ds(0, CHUNK)]],
                              cen_rows, sem).wait()
        pltpu.make_async_copy(tbl.at[pos_idx.at[pl.ds(0, CHUNK)]],
                              pos_rows, sem).wait()
        pltpu.make_async_copy(tbl.at[neg_idx.at[pl.ds(0, NEG_PER_CHUNK)]],
                              neg_rows, sem).wait()

    def compute(c, buf):
        cen_rows, pos_rows, neg_rows, _ = buf

        def item_body(it, carry):
            cvec = [cen_rows[it, pl.ds(16 * j, 16)] for j in range(4)]
            acc = cvec[0] * pos_rows[it, pl.ds(64, 16)]
            for j in range(1, 4):
                acc = acc + cvec[j] * pos_rows[it, pl.ds(64 + 16 * j, 16)]
            pbase = it * SCORES_PER_ITEM * 16
            partials[pl.ds(pbase, 16)] = acc
            for k in range(K_NEG):
                r = it * K_NEG + k
                acc = cvec[0] * neg_rows[r, pl.ds(64, 16)]
                for j in range(1, 4):
                    acc = acc + cvec[j] * neg_rows[r, pl.ds(64 + 16 * j, 16)]
                partials[pl.ds(pbase + (1 + k) * 16, 16)] = -acc
            return carry

        lax.fori_loop(0, CHUNK, item_body, 0, unroll=2)

        ivec = lax.iota(jnp.int32, 16) * 16

        def group_body(grp, carry):
            acc = plsc.load_gather(partials, [ivec + grp * 256])
            for d in range(1, 16):
                acc = acc + plsc.load_gather(partials, [ivec + (grp * 256 + d)])
            scores[pl.ds(grp * 16, 16)] = acc
            return carry

        lax.fori_loop(0, CHUNK_SCORES // 16, group_body, 0, unroll=2)
        pltpu.sync_copy(
            scores,
            scores_out.at[pl.ds(base * SCORES_PER_ITEM + c * CHUNK_SCORES,
                                CHUNK_SCORES)])

    fire(0, bufs[0])

    def pair_body(c2, carry):
        c0 = 2 * c2
        fire(c0 + 1, bufs[1])
        drain(bufs[0])
        compute(c0, bufs[0])

        @pl.when(c2 < N_CHUNKS // 2 - 1)
        def _():
            fire(c0 + 2, bufs[0])

        drain(bufs[1])
        compute(c0 + 1, bufs[1])
        return carry

    lax.fori_loop(0, N_CHUNKS // 2, pair_body, 0)


@functools.partial(
    pl.kernel,
    out_type=jax.ShapeDtypeStruct((BATCH * SCORES_PER_ITEM,), jnp.float32),
    mesh=plsc.VectorSubcoreMesh(core_axis_name="c", subcore_axis_name="s"),
    compiler_params=pltpu.CompilerParams(needs_layout_passes=False,
                                         use_tc_tiling_on_sc=True),
    scratch_types=[
        pltpu.VMEM((B_PER_W,), jnp.int32),
        pltpu.VMEM((B_PER_W,), jnp.int32),
        pltpu.VMEM((NEG_PER_W,), jnp.int32),
        pltpu.VMEM((CHUNK, 128), jnp.float32),
        pltpu.VMEM((CHUNK, 128), jnp.float32),
        pltpu.VMEM((NEG_PER_CHUNK, 128), jnp.float32),
        pltpu.VMEM((CHUNK, 128), jnp.float32),
        pltpu.VMEM((CHUNK, 128), jnp.float32),
        pltpu.VMEM((NEG_PER_CHUNK, 128), jnp.float32),
        pltpu.VMEM((CHUNK_SCORES * 16,), jnp.float32),
        pltpu.VMEM((CHUNK_SCORES,), jnp.float32),
        pltpu.SemaphoreType.DMA,
        pltpu.SemaphoreType.DMA,
    ],
)
def _sc_scores(*args):
    _sc_scores_kernel(*args)


def _tc_loss_kernel(s_ref, o_ref):
    x = s_ref[...]
    y = -jnp.log(jax.nn.sigmoid(x) + 1e-10)
    o_ref[0, 0] = jnp.sum(y) / BATCH


def kernel(center_words, context_words, negative_samples, center_emb,
           context_emb):
    cen_w = center_words.astype(jnp.int32)
    ctx_w = context_words.astype(jnp.int32)
    neg_w = negative_samples.astype(jnp.int32).reshape(-1)
    merged = _repack(center_emb.T, context_emb.T)
    scores = _sc_scores(cen_w, ctx_w, neg_w, merged)
    scores2d = scores.reshape(BATCH * SCORES_PER_ITEM // 128, 128)
    loss = pl.pallas_call(
        _tc_loss_kernel,
        out_shape=jax.ShapeDtypeStruct((1, 1), jnp.float32),
        in_specs=[pl.BlockSpec(memory_space=pltpu.VMEM)],
        out_specs=pl.BlockSpec(memory_space=pltpu.SMEM),
    )(scores2d)
    return loss[0, 0]
